# Initial kernel scaffold; baseline (speedup 1.0000x reference)
import jax
import jax.numpy as jnp
from jax.experimental import pallas as pl


def kernel(mask, emb_weight):
    raise NotImplementedError("write your pallas kernel here")


# speedup vs baseline: 4.2081x; 4.2081x over previous
import jax
import jax.numpy as jnp
from jax.experimental import pallas as pl

B, L, D = 4096, 200, 64
N = B * L

ROWS_PER_BLOCK = 8192
GRID = N // ROWS_PER_BLOCK


def _body(mask_ref, w_ref, out_ref):
    m = mask_ref[...]
    w = w_ref[...]
    out = jnp.where(m == 0, w[0:1, :],
          jnp.where(m == 1, w[1:2, :],
          jnp.where(m == 2, w[2:3, :], w[3:4, :])))
    out_ref[...] = out


def kernel(mask, emb_weight):
    shape = mask.shape
    flat = mask.reshape(N, 1).astype(jnp.int32)
    out = pl.pallas_call(
        _body,
        grid=(GRID,),
        in_specs=[
            pl.BlockSpec((ROWS_PER_BLOCK, 1), lambda g: (g, 0)),
            pl.BlockSpec((4, D), lambda g: (0, 0)),
        ],
        out_specs=pl.BlockSpec((ROWS_PER_BLOCK, D), lambda g: (g, 0)),
        out_shape=jax.ShapeDtypeStruct((N, D), jnp.float32),
    )(flat, emb_weight)
    return out.reshape(shape[0], shape[1], D)
